# Initial kernel scaffold; baseline (speedup 1.0000x reference)
import jax
import jax.numpy as jnp
from jax.experimental import pallas as pl


def kernel(inputs, tables, W, b):
    raise NotImplementedError("write your pallas kernel here")


# speedup vs baseline: 7.3719x; 7.3719x over previous
import functools

import jax
import jax.numpy as jnp
from jax import lax
from jax.experimental import pallas as pl
from jax.experimental.pallas import tpu as pltpu, tpu_sc as plsc

_NUM_TABLES = 26
_VOCAB = 100000
_EMBED_DIM = 32
_BATCH = 16384
_DENSE_OUT = 5

_ROWS = _NUM_TABLES * _BATCH


def _sc_gather(table_flat, idx_flat):
    info = plsc.get_sparse_core_info()
    nc, ns = info.num_cores, info.num_subcores
    nw = nc * ns
    rows_per_w = _ROWS // nw
    chunk = 1024
    n_chunks = rows_per_w // chunk
    assert rows_per_w % chunk == 0

    mesh = plsc.VectorSubcoreMesh(core_axis_name="c", subcore_axis_name="s")

    @functools.partial(
        pl.kernel,
        mesh=mesh,
        out_type=jax.ShapeDtypeStruct((_ROWS, _EMBED_DIM), jnp.float32),
        scratch_types=[
            pltpu.VMEM((chunk,), jnp.int32),
            pltpu.VMEM((chunk, _EMBED_DIM), jnp.float32),
            pltpu.SemaphoreType.DMA,
        ],
        compiler_params=pltpu.CompilerParams(use_tc_tiling_on_sc=False),
    )
    def k(table_hbm, idx_hbm, out_hbm, idx_v, rows_v, sem):
        wid = lax.axis_index("s") * nc + lax.axis_index("c")
        base = wid * rows_per_w
        for c in range(n_chunks):
            off = base + c * chunk
            pltpu.sync_copy(idx_hbm.at[pl.ds(off, chunk)], idx_v)
            pltpu.async_copy(table_hbm.at[idx_v], rows_v, sem).wait()
            pltpu.sync_copy(rows_v, out_hbm.at[pl.ds(off, chunk)])

    return k(table_flat, idx_flat)


def _tc_dense(emb, w3, b2):
    bm = 2048
    grid = _BATCH // bm

    def mm(emb_ref, w_ref, b_ref, out_ref):
        acc = jnp.zeros((bm, _DENSE_OUT), jnp.float32)
        for t in range(_NUM_TABLES):
            acc += jnp.dot(emb_ref[t], w_ref[t],
                           preferred_element_type=jnp.float32)
        out_ref[...] = acc + b_ref[...]

    return pl.pallas_call(
        mm,
        grid=(grid,),
        in_specs=[
            pl.BlockSpec((_NUM_TABLES, bm, _EMBED_DIM), lambda i: (0, i, 0)),
            pl.BlockSpec((_NUM_TABLES, _EMBED_DIM, _DENSE_OUT),
                         lambda i: (0, 0, 0)),
            pl.BlockSpec((1, _DENSE_OUT), lambda i: (0, 0)),
        ],
        out_specs=pl.BlockSpec((bm, _DENSE_OUT), lambda i: (i, 0)),
        out_shape=jax.ShapeDtypeStruct((_BATCH, _DENSE_OUT), jnp.float32),
    )(emb, w3, b2)


def kernel(inputs, tables, W, b):
    offs = (jnp.arange(_NUM_TABLES, dtype=jnp.int32) * _VOCAB)[:, None]
    idx_flat = (inputs + offs).reshape(_ROWS)
    table_flat = tables.reshape(_NUM_TABLES * _VOCAB, _EMBED_DIM)

    emb_flat = _sc_gather(table_flat, idx_flat)

    emb = emb_flat.reshape(_NUM_TABLES, _BATCH, _EMBED_DIM)
    w3 = W.reshape(_NUM_TABLES, _EMBED_DIM, _DENSE_OUT)
    b2 = b.reshape(1, _DENSE_OUT)
    return _tc_dense(emb, w3, b2)


# speedup vs baseline: 22.0219x; 2.9873x over previous
import functools

import jax
import jax.numpy as jnp
from jax import lax
from jax.experimental import pallas as pl
from jax.experimental.pallas import tpu as pltpu, tpu_sc as plsc

_NT = 26
_V = 100000
_D = 32
_B = 16384
_O = 5
_K = _NT * _D
_RB = _B // 128


def _sc_stripe_gather(tabT, idx1):
    mesh = plsc.VectorSubcoreMesh(core_axis_name="c", subcore_axis_name="s")

    @functools.partial(
        pl.kernel,
        mesh=mesh,
        out_type=jax.ShapeDtypeStruct((_K, _RB, 128), jnp.float32),
        scratch_types=[
            pltpu.VMEM((_V,), jnp.float32),
            pltpu.VMEM((_B,), jnp.int32),
            pltpu.VMEM((32, 128), jnp.float32),
            pltpu.SemaphoreType.DMA,
        ],
        compiler_params=pltpu.CompilerParams(
            use_tc_tiling_on_sc=True, needs_layout_passes=False
        ),
    )
    def k(tab_hbm, idx_hbm, out_hbm, stripe_v, idx_v, out_v, sem):
        wid = lax.axis_index("s") * 2 + lax.axis_index("c")
        for t in range(_NT):
            pltpu.sync_copy(tab_hbm.at[t, wid, :], stripe_v)
            pltpu.sync_copy(idx_hbm.at[pl.ds(t * _B, _B)], idx_v)

            def chunk_body(c, _, t=t):
                def gat_body(i, _):
                    vidx = idx_v[pl.ds(c * 4096 + i * 16, 16)]
                    vals = plsc.load_gather(stripe_v, [vidx])
                    out_v[i // 8, pl.ds((i % 8) * 16, 16)] = vals
                    return 0

                lax.fori_loop(0, 256, gat_body, 0)
                pltpu.sync_copy(
                    out_v, out_hbm.at[t * _D + wid, pl.ds(c * 32, 32), :]
                )
                return 0

            lax.fori_loop(0, 4, chunk_body, 0)

    return k(tabT, idx1)


def _tc_dense(emb3, W, b2):
    R = 16
    grid = _RB // R

    def mm(emb_ref, w_ref, b_ref, out_ref):
        w = w_ref[...]
        bias = b_ref[...]
        for j in range(R):
            x = emb_ref[:, j, :]
            acc = lax.dot_general(
                x, w, (((0,), (0,)), ((), ())),
                preferred_element_type=jnp.float32,
            )
            out_ref[j] = acc + bias

    return pl.pallas_call(
        mm,
        grid=(grid,),
        in_specs=[
            pl.BlockSpec((_K, R, 128), lambda i: (0, i, 0)),
            pl.BlockSpec((_K, _O), lambda i: (0, 0)),
            pl.BlockSpec((1, _O), lambda i: (0, 0)),
        ],
        out_specs=pl.BlockSpec((R, 128, _O), lambda i: (i, 0, 0)),
        out_shape=jax.ShapeDtypeStruct((_RB, 128, _O), jnp.float32),
    )(emb3, W, b2)


def kernel(inputs, tables, W, b):
    tabT = jnp.transpose(tables, (0, 2, 1))
    idx1 = inputs.reshape(_NT * _B)
    emb3 = _sc_stripe_gather(tabT, idx1)
    out3 = _tc_dense(emb3, W, b.reshape(1, _O))
    return out3.reshape(_B, _O)


# speedup vs baseline: 29.8701x; 1.3564x over previous
import functools

import jax
import jax.numpy as jnp
from jax import lax
from jax.experimental import pallas as pl
from jax.experimental.pallas import tpu as pltpu, tpu_sc as plsc

_NT = 26
_V = 100000
_D = 32
_B = 16384
_O = 5
_K = _NT * _D
_RB = _B // 128


def _sc_stripe_gather(tabT, idx1):
    mesh = plsc.VectorSubcoreMesh(core_axis_name="c", subcore_axis_name="s")

    @functools.partial(
        pl.kernel,
        mesh=mesh,
        out_type=jax.ShapeDtypeStruct((_K, _RB, 128), jnp.float32),
        scratch_types=[
            pltpu.VMEM((_V,), jnp.float32),
            pltpu.VMEM((_B // 2,), jnp.int32),
            [pltpu.VMEM((32, 128), jnp.float32) for _ in range(4)],
            pltpu.SemaphoreType.DMA,
            pltpu.SemaphoreType.DMA,
            pltpu.SemaphoreType.DMA,
        ],
        compiler_params=pltpu.CompilerParams(
            use_tc_tiling_on_sc=True, needs_layout_passes=False
        ),
    )
    def k(tab_hbm, idx_hbm, out_hbm, stripe_v, idx_v, out_vs, sem_s, sem_i,
          sem_o):
        wid = lax.axis_index("s") * 2 + lax.axis_index("c")

        def table_body(t, _):
            cp_s = pltpu.async_copy(tab_hbm.at[t, wid, :], stripe_v, sem_s)
            cp_i = pltpu.async_copy(
                idx_hbm.at[pl.ds(t * _B, _B // 2)], idx_v, sem_i
            )
            cp_s.wait()
            cp_i.wait()
            out_cps = []
            for c in range(4):
                if c == 2:
                    pltpu.sync_copy(
                        idx_hbm.at[pl.ds(t * _B + _B // 2, _B // 2)], idx_v
                    )
                ov = out_vs[c]

                def gat_body(r, _, c=c, ov=ov):
                    for j in range(8):
                        off = (c % 2) * 4096 + r * 128 + j * 16
                        vidx = idx_v[pl.ds(off, 16)]
                        vals = plsc.load_gather(stripe_v, [vidx])
                        ov[r, pl.ds(j * 16, 16)] = vals
                    return 0

                lax.fori_loop(0, 32, gat_body, 0)
                out_cps.append(
                    pltpu.async_copy(
                        ov,
                        out_hbm.at[t * _D + wid, pl.ds(c * 32, 32), :],
                        sem_o,
                    )
                )
            for cp in out_cps:
                cp.wait()
            return 0

        lax.fori_loop(0, _NT, table_body, 0)

    return k(tabT, idx1)


def _tc_dense(emb3, W, b2):
    R = 16
    grid = _RB // R

    def mm(emb_ref, w_ref, b_ref, out_ref):
        w = w_ref[...]
        bias = b_ref[...]
        for j in range(R):
            x = emb_ref[:, j, :]
            acc = lax.dot_general(
                x, w, (((0,), (0,)), ((), ())),
                preferred_element_type=jnp.float32,
            )
            out_ref[j] = acc + bias

    return pl.pallas_call(
        mm,
        grid=(grid,),
        in_specs=[
            pl.BlockSpec((_K, R, 128), lambda i: (0, i, 0)),
            pl.BlockSpec((_K, _O), lambda i: (0, 0)),
            pl.BlockSpec((1, _O), lambda i: (0, 0)),
        ],
        out_specs=pl.BlockSpec((R, 128, _O), lambda i: (i, 0, 0)),
        out_shape=jax.ShapeDtypeStruct((_RB, 128, _O), jnp.float32),
    )(emb3, W, b2)


def kernel(inputs, tables, W, b):
    tabT = jnp.transpose(tables, (0, 2, 1))
    idx1 = inputs.reshape(_NT * _B)
    emb3 = _sc_stripe_gather(tabT, idx1)
    out3 = _tc_dense(emb3, W, b.reshape(1, _O))
    return out3.reshape(_B, _O)


# speedup vs baseline: 31.9147x; 1.0684x over previous
import functools

import jax
import jax.numpy as jnp
from jax import lax
from jax.experimental import pallas as pl
from jax.experimental.pallas import tpu as pltpu, tpu_sc as plsc

_NT = 26
_V = 100000
_D = 32
_B = 16384
_O = 5
_K = _NT * _D
_RB = _B // 128


def _sc_stripe_gather(tabT, idx1):
    mesh = plsc.VectorSubcoreMesh(core_axis_name="c", subcore_axis_name="s")

    @functools.partial(
        pl.kernel,
        mesh=mesh,
        out_type=jax.ShapeDtypeStruct((_K, _RB, 128), jnp.float32),
        scratch_types=[
            pltpu.VMEM((_V,), jnp.float32),
            [pltpu.VMEM((_B // 2,), jnp.int32) for _ in range(2)],
            [pltpu.VMEM((32, 128), jnp.float32) for _ in range(2)],
            pltpu.SemaphoreType.DMA,
            pltpu.SemaphoreType.DMA,
            pltpu.SemaphoreType.DMA,
        ],
        compiler_params=pltpu.CompilerParams(
            use_tc_tiling_on_sc=True, needs_layout_passes=False
        ),
    )
    def k(tab_hbm, idx_hbm, out_hbm, stripe_v, idx_vs, out_vs, sem_s, sem_i,
          sem_o):
        wid = lax.axis_index("s") * 2 + lax.axis_index("c")

        def drain_out(n):
            for _ in range(n):
                pltpu.make_async_copy(
                    out_hbm.at[0, pl.ds(0, 32), :], out_vs[0], sem_o
                ).wait()

        def table_body(t, _):
            cp_s = pltpu.async_copy(tab_hbm.at[t, wid, :], stripe_v, sem_s)
            cp_i0 = pltpu.async_copy(
                idx_hbm.at[pl.ds(t * _B, _B // 2)], idx_vs[0], sem_i
            )
            cp_i1 = pltpu.async_copy(
                idx_hbm.at[pl.ds(t * _B + _B // 2, _B // 2)], idx_vs[1], sem_i
            )
            @pl.when(t > 0)
            def _():
                drain_out(2)

            cp_s.wait()
            cp_i0.wait()
            out_cps = []
            for c in range(4):
                if c == 2:
                    cp_i1.wait()
                if c >= 2:
                    out_cps[c - 2].wait()
                iv = idx_vs[c // 2]
                ov = out_vs[c % 2]

                def gat_body(r, _, c=c, iv=iv, ov=ov):
                    for j in range(8):
                        off = (c % 2) * 4096 + r * 128 + j * 16
                        vidx = iv[pl.ds(off, 16)]
                        vals = plsc.load_gather(stripe_v, [vidx])
                        ov[r, pl.ds(j * 16, 16)] = vals
                    return 0

                lax.fori_loop(0, 32, gat_body, 0)
                out_cps.append(
                    pltpu.async_copy(
                        ov,
                        out_hbm.at[t * _D + wid, pl.ds(c * 32, 32), :],
                        sem_o,
                    )
                )
            return 0

        lax.fori_loop(0, _NT, table_body, 0)
        drain_out(2)

    return k(tabT, idx1)


def _tc_dense(emb3, W, b2):
    R = 16
    grid = _RB // R

    def mm(emb_ref, w_ref, b_ref, out_ref):
        x = emb_ref[...].reshape(_K, R * 128)
        acc = lax.dot_general(
            x, w_ref[...], (((0,), (0,)), ((), ())),
            preferred_element_type=jnp.float32,
        )
        out_ref[...] = acc.reshape(R, 128, _O) + b_ref[...][None]

    return pl.pallas_call(
        mm,
        grid=(grid,),
        in_specs=[
            pl.BlockSpec((_K, R, 128), lambda i: (0, i, 0)),
            pl.BlockSpec((_K, _O), lambda i: (0, 0)),
            pl.BlockSpec((1, _O), lambda i: (0, 0)),
        ],
        out_specs=pl.BlockSpec((R, 128, _O), lambda i: (i, 0, 0)),
        out_shape=jax.ShapeDtypeStruct((_RB, 128, _O), jnp.float32),
    )(emb3, W, b2)


def kernel(inputs, tables, W, b):
    tabT = jnp.transpose(tables, (0, 2, 1))
    idx1 = inputs.reshape(_NT * _B)
    emb3 = _sc_stripe_gather(tabT, idx1)
    out3 = _tc_dense(emb3, W, b.reshape(1, _O))
    return out3.reshape(_B, _O)


# speedup vs baseline: 32.1362x; 1.0069x over previous
import functools

import jax
import jax.numpy as jnp
from jax import lax
from jax.experimental import pallas as pl
from jax.experimental.pallas import tpu as pltpu, tpu_sc as plsc

_NT = 26
_V = 100000
_D = 32
_B = 16384
_O = 5
_K = _NT * _D
_RB = _B // 128


def _sc_stripe_gather(tabT, idx1):
    mesh = plsc.VectorSubcoreMesh(core_axis_name="c", subcore_axis_name="s")

    @functools.partial(
        pl.kernel,
        mesh=mesh,
        out_type=jax.ShapeDtypeStruct((_K, _RB, 128), jnp.float32),
        scratch_types=[
            pltpu.VMEM((_V,), jnp.float32),
            [pltpu.VMEM((_B // 2,), jnp.int32) for _ in range(2)],
            [pltpu.VMEM((32, 128), jnp.float32) for _ in range(2)],
            pltpu.SemaphoreType.DMA,
            pltpu.SemaphoreType.DMA,
            pltpu.SemaphoreType.DMA,
        ],
        compiler_params=pltpu.CompilerParams(
            use_tc_tiling_on_sc=True, needs_layout_passes=False
        ),
    )
    def k(tab_hbm, idx_hbm, out_hbm, stripe_v, idx_vs, out_vs, sem_s, sem_i,
          sem_o):
        wid = lax.axis_index("s") * 2 + lax.axis_index("c")

        def drain_out(n):
            for _ in range(n):
                pltpu.make_async_copy(
                    out_hbm.at[0, pl.ds(0, 32), :], out_vs[0], sem_o
                ).wait()

        def table_body(t, _):
            cp_s = pltpu.async_copy(tab_hbm.at[t, wid, :], stripe_v, sem_s)
            cp_i0 = pltpu.async_copy(
                idx_hbm.at[pl.ds(t * _B, _B // 2)], idx_vs[0], sem_i
            )
            cp_i1 = pltpu.async_copy(
                idx_hbm.at[pl.ds(t * _B + _B // 2, _B // 2)], idx_vs[1], sem_i
            )
            @pl.when(t > 0)
            def _():
                drain_out(2)

            cp_s.wait()
            cp_i0.wait()
            out_cps = []
            for c in range(4):
                if c == 2:
                    cp_i1.wait()
                if c >= 2:
                    out_cps[c - 2].wait()
                iv = idx_vs[c // 2]
                ov = out_vs[c % 2]

                def gat_body(r2, _, c=c, iv=iv, ov=ov):
                    for j in range(16):
                        off = (c % 2) * 4096 + r2 * 256 + j * 16
                        vidx = iv[pl.ds(off, 16)]
                        vals = plsc.load_gather(stripe_v, [vidx])
                        ov[r2 * 2 + j // 8, pl.ds((j % 8) * 16, 16)] = vals
                    return 0

                lax.fori_loop(0, 16, gat_body, 0)
                out_cps.append(
                    pltpu.async_copy(
                        ov,
                        out_hbm.at[t * _D + wid, pl.ds(c * 32, 32), :],
                        sem_o,
                    )
                )
            return 0

        lax.fori_loop(0, _NT, table_body, 0)
        drain_out(2)

    return k(tabT, idx1)


def _tc_dense(emb3, W, b2):
    R = 16
    grid = _RB // R

    def mm(emb_ref, w_ref, b_ref, out_ref):
        x = emb_ref[...].reshape(_K, R * 128)
        acc = lax.dot_general(
            x, w_ref[...], (((0,), (0,)), ((), ())),
            preferred_element_type=jnp.float32,
        )
        out_ref[...] = acc.reshape(R, 128, _O) + b_ref[...][None]

    return pl.pallas_call(
        mm,
        grid=(grid,),
        in_specs=[
            pl.BlockSpec((_K, R, 128), lambda i: (0, i, 0)),
            pl.BlockSpec((_K, _O), lambda i: (0, 0)),
            pl.BlockSpec((1, _O), lambda i: (0, 0)),
        ],
        out_specs=pl.BlockSpec((R, 128, _O), lambda i: (i, 0, 0)),
        out_shape=jax.ShapeDtypeStruct((_RB, 128, _O), jnp.float32),
    )(emb3, W, b2)


def kernel(inputs, tables, W, b):
    tabT = jnp.transpose(tables, (0, 2, 1))
    idx1 = inputs.reshape(_NT * _B)
    emb3 = _sc_stripe_gather(tabT, idx1)
    out3 = _tc_dense(emb3, W, b.reshape(1, _O))
    return out3.reshape(_B, _O)


# speedup vs baseline: 33.2440x; 1.0345x over previous
import functools

import jax
import jax.numpy as jnp
from jax import lax
from jax.experimental import pallas as pl
from jax.experimental.pallas import tpu as pltpu, tpu_sc as plsc

_NT = 26
_V = 100000
_D = 32
_B = 16384
_O = 5
_K = _NT * _D
_RB = _B // 128


def _sc_stripe_gather(tabT, idx1):
    mesh = plsc.VectorSubcoreMesh(core_axis_name="c", subcore_axis_name="s")

    @functools.partial(
        pl.kernel,
        mesh=mesh,
        out_type=jax.ShapeDtypeStruct((_K, _RB, 128), jnp.float32),
        scratch_types=[
            pltpu.VMEM((_V,), jnp.float32),
            [pltpu.VMEM((_B // 2,), jnp.int32) for _ in range(2)],
            [pltpu.VMEM((32, 128), jnp.float32) for _ in range(2)],
            pltpu.SemaphoreType.DMA,
            pltpu.SemaphoreType.DMA,
            pltpu.SemaphoreType.DMA,
        ],
        compiler_params=pltpu.CompilerParams(
            use_tc_tiling_on_sc=True, needs_layout_passes=False
        ),
    )
    def k(tab_hbm, idx_hbm, out_hbm, stripe_v, idx_vs, out_vs, sem_s, sem_i,
          sem_o):
        wid = lax.axis_index("s") * 2 + lax.axis_index("c")

        def drain_out(n):
            for _ in range(n):
                pltpu.make_async_copy(
                    out_hbm.at[0, pl.ds(0, 32), :], out_vs[0], sem_o
                ).wait()

        def table_body(t, _):
            cp_s = pltpu.async_copy(tab_hbm.at[t, wid, :], stripe_v, sem_s)
            cp_i0 = pltpu.async_copy(
                idx_hbm.at[pl.ds(t * _B, _B // 2)], idx_vs[0], sem_i
            )
            cp_i1 = pltpu.async_copy(
                idx_hbm.at[pl.ds(t * _B + _B // 2, _B // 2)], idx_vs[1], sem_i
            )
            @pl.when(t > 0)
            def _():
                drain_out(2)

            cp_s.wait()
            cp_i0.wait()
            out_cps = []
            for c in range(4):
                if c == 2:
                    cp_i1.wait()
                if c >= 2:
                    out_cps[c - 2].wait()
                iv = idx_vs[c // 2]
                ov = out_vs[c % 2]

                def gat_body(r2, _, c=c, iv=iv, ov=ov):
                    for j in range(16):
                        off = (c % 2) * 4096 + r2 * 256 + j * 16
                        vidx = iv[pl.ds(off, 16)]
                        vals = plsc.load_gather(stripe_v, [vidx])
                        ov[r2 * 2 + j // 8, pl.ds((j % 8) * 16, 16)] = vals
                    return 0

                lax.fori_loop(0, 16, gat_body, 0)
                out_cps.append(
                    pltpu.async_copy(
                        ov,
                        out_hbm.at[t * _D + wid, pl.ds(c * 32, 32), :],
                        sem_o,
                    )
                )
            return 0

        lax.fori_loop(0, _NT, table_body, 0)
        drain_out(2)

    return k(tabT, idx1)


def _tc_dense(emb3, wT, bT):
    R = 16
    grid = _RB // R

    def mm(emb_ref, w_ref, b_ref, out_ref):
        x = emb_ref[...].reshape(_K, R * 128)
        acc = lax.dot_general(
            w_ref[...], x, (((1,), (0,)), ((), ())),
            preferred_element_type=jnp.float32,
        )
        out_ref[...] = acc + b_ref[...]

    return pl.pallas_call(
        mm,
        grid=(grid,),
        in_specs=[
            pl.BlockSpec((_K, R, 128), lambda i: (0, i, 0)),
            pl.BlockSpec((_O, _K), lambda i: (0, 0)),
            pl.BlockSpec((_O, 1), lambda i: (0, 0)),
        ],
        out_specs=pl.BlockSpec((_O, R * 128), lambda i: (0, i)),
        out_shape=jax.ShapeDtypeStruct((_O, _B), jnp.float32),
    )(emb3, wT, bT)


def kernel(inputs, tables, W, b):
    tabT = jnp.transpose(tables, (0, 2, 1))
    idx1 = inputs.reshape(_NT * _B)
    emb3 = _sc_stripe_gather(tabT, idx1)
    outT = _tc_dense(emb3, W.T, b.reshape(_O, 1))
    return outT.T
